# baseline (speedup 1.0000x reference)
import functools

import jax
import jax.numpy as jnp
from jax import lax
from jax.experimental import pallas as pl
from jax.experimental.pallas import tpu as pltpu
from jax.experimental.pallas import tpu_sc as plsc

N_NODES = 10000
K_PROP = 10
ALPHA = 0.1
F = 16
N_SUB = 16
N_CORE = 2
N_PAD = 10112
NPT = N_PAD // N_SUB
E = 320000
CHUNK = 512
CPT = 20
E_PAD = N_CORE * N_SUB * CPT * CHUNK


def _mlp_body(x_ref, w1_ref, b1_ref, w2_ref, b2_ref, o_ref):
  x = x_ref[...]
  g = lax.dot_general(x, w1_ref[...], (((1,), (1,)), ((), ())),
                      preferred_element_type=jnp.float32)
  g = jnp.maximum(g + b1_ref[...], 0.0)
  h = lax.dot_general(g, w2_ref[...], (((1,), (1,)), ((), ())),
                      preferred_element_type=jnp.float32)
  o_ref[...] = h + b2_ref[...]


def _mlp(x, W1, b1, W2, b2):
  n = x.shape[0]
  blk = 1000
  return pl.pallas_call(
      _mlp_body,
      grid=(n // blk,),
      in_specs=[
          pl.BlockSpec((blk, 128), lambda i: (i, 0)),
          pl.BlockSpec((64, 128), lambda i: (0, 0)),
          pl.BlockSpec((1, 64), lambda i: (0, 0)),
          pl.BlockSpec((16, 64), lambda i: (0, 0)),
          pl.BlockSpec((1, 16), lambda i: (0, 0)),
      ],
      out_specs=pl.BlockSpec((blk, 16), lambda i: (i, 0)),
      out_shape=jax.ShapeDtypeStruct((n, 16), jnp.float32),
  )(x, W1, b1.reshape(1, 64), W2, b2.reshape(1, 16))


def _lsm_body(x_ref, o_ref):
  x = x_ref[...]
  m = jnp.max(x, axis=1, keepdims=True)
  xm = x - m
  lse = jnp.log(jnp.sum(jnp.exp(xm), axis=1, keepdims=True))
  o_ref[...] = xm - lse


def _log_softmax(x):
  n = x.shape[0]
  blk = 1000
  return pl.pallas_call(
      _lsm_body,
      grid=(n // blk,),
      in_specs=[pl.BlockSpec((blk, 16), lambda i: (i, 0))],
      out_specs=pl.BlockSpec((blk, 16), lambda i: (i, 0)),
      out_shape=jax.ShapeDtypeStruct((n, 16), jnp.float32),
  )(x)


def _rsqrt16(d):
  y = jnp.full((F,), 1.0, jnp.float32)
  for k in range(1, 11):
    y = jnp.where(d >= float(4 ** k) * 0.5, float(2.0 ** -k), y)
  for k in range(1, 11):
    y = jnp.where(d <= float(4 ** -k) * 2.0, float(2.0 ** k), y)
  for _ in range(6):
    y = y * (1.5 - 0.5 * d * y * y)
  return y


def _sc_body(rows_hbm, cols_hbm, h_hbm, out_hbm, p_hbm,
             T, S, ab, bb, tb, sb, pb, zb, rows_v, cols_v,
             msga, msgb, gsem, ssem, xsem):
  c = lax.axis_index("c")
  w = lax.axis_index("s")
  nbase = w * NPT
  nsl = pl.ds(nbase, NPT)

  def _exchange(par):
    pltpu.sync_copy(S.at[nsl], sb)
    pltpu.sync_copy(sb, p_hbm.at[par, c, nsl])
    pltpu.sync_copy(zb, S.at[nsl])
    plsc.subcore_barrier()

    @pl.when(w == 0)
    def _():
      pltpu.semaphore_signal(xsem, 1, core_index=1 - c)
      pltpu.semaphore_wait(xsem, 1)
    plsc.subcore_barrier()
    pltpu.sync_copy(p_hbm.at[par, 1 - c, nsl], pb)

  pltpu.sync_copy(rows_hbm.at[c, w], rows_v)
  pltpu.sync_copy(cols_hbm.at[c, w], cols_v)
  pltpu.sync_copy(h_hbm.at[nsl], tb)

  def _fill(i, _):
    zb[i] = jnp.zeros((F,), jnp.float32)
    return 0
  lax.fori_loop(0, NPT, _fill, 0)

  def _fill1(i, _):
    msga[i] = jnp.full((F,), 1.0, jnp.float32)
    return 0
  lax.fori_loop(0, CHUNK, _fill1, 0)

  pltpu.sync_copy(zb, S.at[nsl])
  plsc.subcore_barrier()

  def _deg(ch, _):
    pltpu.sync_copy(msga, S.at[cols_v.at[ch]], add=True)
    return 0
  lax.fori_loop(0, CPT, _deg, 0)
  plsc.subcore_barrier()
  _exchange(1)

  def _const(i, _):
    deg = sb[i] + pb[i] + 1.0
    dv = _rsqrt16(deg)
    h = tb[i]
    ab[i] = (1.0 - ALPHA) * dv * dv
    bb[i] = ALPHA * dv * h
    tb[i] = dv * h
    return 0
  lax.fori_loop(0, NPT, _const, 0)
  pltpu.sync_copy(tb, T.at[nsl])
  plsc.subcore_barrier()

  def _gstart(ch, buf):
    pltpu.make_async_copy(T.at[rows_v.at[ch]], buf, gsem).start()

  def _gwait(ch, buf):
    pltpu.make_async_copy(T.at[rows_v.at[ch]], buf, gsem).wait()

  def _sstart(ch, buf):
    pltpu.make_async_copy(buf, S.at[cols_v.at[ch]], ssem).start(add=True)

  def _swait(ch, buf):
    pltpu.make_async_copy(buf, S.at[cols_v.at[ch]], ssem).wait()

  def _step(k, carry):
    _gstart(0, msga)

    def _pipe(j, c2):
      chA = 2 * j
      chB = chA + 1
      _gwait(chA, msga)
      _gstart(chB, msgb)
      _sstart(chA, msga)
      _gwait(chB, msgb)
      _swait(chA, msga)

      @pl.when(j < CPT // 2 - 1)
      def _():
        _gstart(chA + 2, msga)

      _sstart(chB, msgb)
      _swait(chB, msgb)
      return c2
    lax.fori_loop(0, CPT // 2, _pipe, 0)
    plsc.subcore_barrier()

    _exchange(k & 1)

    def _upd(i, c2):
      for u in range(4):
        q = i * 4 + u
        tb[q] = ab[q] * (sb[q] + pb[q] + tb[q]) + bb[q]
      return c2
    lax.fori_loop(0, NPT // 4, _upd, 0)
    pltpu.sync_copy(tb, T.at[nsl])
    plsc.subcore_barrier()
    return carry
  lax.fori_loop(0, K_PROP, _step, 0)

  def _fin(i, _):
    sb[i] = tb[i] * _rsqrt16(ab[i] * (1.0 / (1.0 - ALPHA)))
    return 0
  lax.fori_loop(0, NPT, _fin, 0)

  @pl.when(c == 0)
  def _():
    pltpu.sync_copy(sb, out_hbm.at[nsl])


_sc_prop = functools.partial(
    pl.kernel,
    out_type=(
        jax.ShapeDtypeStruct((N_PAD, F), jnp.float32),
        jax.ShapeDtypeStruct((2, N_CORE, N_PAD, F), jnp.float32),
    ),
    mesh=plsc.VectorSubcoreMesh(
        core_axis_name="c", subcore_axis_name="s", num_cores=2),
    compiler_params=pltpu.CompilerParams(
        use_tc_tiling_on_sc=False, needs_layout_passes=False),
    scratch_types=[
        pltpu.VMEM_SHARED((N_PAD, F), jnp.float32),
        pltpu.VMEM_SHARED((N_PAD, F), jnp.float32),
        pltpu.VMEM((NPT, F), jnp.float32),
        pltpu.VMEM((NPT, F), jnp.float32),
        pltpu.VMEM((NPT, F), jnp.float32),
        pltpu.VMEM((NPT, F), jnp.float32),
        pltpu.VMEM((NPT, F), jnp.float32),
        pltpu.VMEM((NPT, F), jnp.float32),
        pltpu.VMEM((CPT, CHUNK), jnp.int32),
        pltpu.VMEM((CPT, CHUNK), jnp.int32),
        pltpu.VMEM((CHUNK, F), jnp.float32),
        pltpu.VMEM((CHUNK, F), jnp.float32),
        pltpu.SemaphoreType.DMA,
        pltpu.SemaphoreType.DMA,
        pltpu.SemaphoreType.REGULAR,
    ],
)(_sc_body)


def kernel(x, edge_index, W1, b1, W2, b2):
  h = _mlp(x, W1, b1, W2, b2)
  h_pad = jnp.pad(h, ((0, N_PAD - N_NODES), (0, 0)))

  ei = edge_index.astype(jnp.int32)
  pad = jnp.full((E_PAD - E,), N_NODES, jnp.int32)
  rows4 = jnp.concatenate([ei[0], pad]).reshape(N_CORE, N_SUB, CPT, CHUNK)
  cols4 = jnp.concatenate([ei[1], pad]).reshape(N_CORE, N_SUB, CPT, CHUNK)

  out, _ = _sc_prop(rows4, cols4, h_pad)
  return _log_softmax(out[:N_NODES])


# speedup vs baseline: 1.0185x; 1.0185x over previous
import functools

import jax
import jax.numpy as jnp
from jax import lax
from jax.experimental import pallas as pl
from jax.experimental.pallas import tpu as pltpu
from jax.experimental.pallas import tpu_sc as plsc

N_NODES = 10000
K_PROP = 10
ALPHA = 0.1
F = 16
N_SUB = 16
N_CORE = 2
N_PAD = 10112
NPT = N_PAD // N_SUB
E = 320000
CHUNK = 256
CPT = 40
E_PAD = N_CORE * N_SUB * CPT * CHUNK


def _mlp_body(x_ref, w1_ref, b1_ref, w2_ref, b2_ref, o_ref):
  x = x_ref[...]
  g = lax.dot_general(x, w1_ref[...], (((1,), (1,)), ((), ())),
                      preferred_element_type=jnp.float32)
  g = jnp.maximum(g + b1_ref[...], 0.0)
  h = lax.dot_general(g, w2_ref[...], (((1,), (1,)), ((), ())),
                      preferred_element_type=jnp.float32)
  o_ref[...] = h + b2_ref[...]


def _mlp(x, W1, b1, W2, b2):
  n = x.shape[0]
  blk = 1000
  return pl.pallas_call(
      _mlp_body,
      grid=(n // blk,),
      in_specs=[
          pl.BlockSpec((blk, 128), lambda i: (i, 0)),
          pl.BlockSpec((64, 128), lambda i: (0, 0)),
          pl.BlockSpec((1, 64), lambda i: (0, 0)),
          pl.BlockSpec((16, 64), lambda i: (0, 0)),
          pl.BlockSpec((1, 16), lambda i: (0, 0)),
      ],
      out_specs=pl.BlockSpec((blk, 16), lambda i: (i, 0)),
      out_shape=jax.ShapeDtypeStruct((n, 16), jnp.float32),
  )(x, W1, b1.reshape(1, 64), W2, b2.reshape(1, 16))


def _lsm_body(x_ref, o_ref):
  x = x_ref[...]
  m = jnp.max(x, axis=1, keepdims=True)
  xm = x - m
  lse = jnp.log(jnp.sum(jnp.exp(xm), axis=1, keepdims=True))
  o_ref[...] = xm - lse


def _log_softmax(x):
  n = x.shape[0]
  blk = 1000
  return pl.pallas_call(
      _lsm_body,
      grid=(n // blk,),
      in_specs=[pl.BlockSpec((blk, 16), lambda i: (i, 0))],
      out_specs=pl.BlockSpec((blk, 16), lambda i: (i, 0)),
      out_shape=jax.ShapeDtypeStruct((n, 16), jnp.float32),
  )(x)


def _rsqrt16(d):
  y = jnp.full((F,), 1.0, jnp.float32)
  for k in range(1, 11):
    y = jnp.where(d >= float(4 ** k) * 0.5, float(2.0 ** -k), y)
  for k in range(1, 11):
    y = jnp.where(d <= float(4 ** -k) * 2.0, float(2.0 ** k), y)
  for _ in range(6):
    y = y * (1.5 - 0.5 * d * y * y)
  return y


def _sc_body(rows_hbm, cols_hbm, h_hbm, out_hbm, p_hbm,
             T, S, ab, bb, tb, sb, pb, zb, rows_v, cols_v,
             msga, msgb, gsem, ssem, xsem, xdsem):
  c = lax.axis_index("c")
  w = lax.axis_index("s")
  nbase = w * NPT
  nsl = pl.ds(nbase, NPT)

  def _exchange(par):
    pltpu.sync_copy(S.at[nsl], sb)
    pltpu.make_async_copy(sb, p_hbm.at[par, c, nsl], xdsem).start()
    pltpu.sync_copy(zb, S.at[nsl])
    pltpu.make_async_copy(sb, p_hbm.at[par, c, nsl], xdsem).wait()
    plsc.subcore_barrier()

    @pl.when(w == 0)
    def _():
      pltpu.semaphore_signal(xsem, 1, core_index=1 - c)
      pltpu.semaphore_wait(xsem, 1)
    plsc.subcore_barrier()
    pltpu.sync_copy(p_hbm.at[par, 1 - c, nsl], pb)

  pltpu.sync_copy(rows_hbm.at[c, w], rows_v)
  pltpu.sync_copy(cols_hbm.at[c, w], cols_v)
  pltpu.sync_copy(h_hbm.at[nsl], tb)

  def _fill(i, _):
    zb[i] = jnp.zeros((F,), jnp.float32)
    return 0
  lax.fori_loop(0, NPT, _fill, 0)

  def _fill1(i, _):
    msga[i] = jnp.full((F,), 1.0, jnp.float32)
    return 0
  lax.fori_loop(0, CHUNK, _fill1, 0)

  pltpu.sync_copy(zb, S.at[nsl])
  plsc.subcore_barrier()

  def _deg(ch, _):
    pltpu.sync_copy(msga, S.at[cols_v.at[ch]], add=True)
    return 0
  lax.fori_loop(0, CPT, _deg, 0)
  plsc.subcore_barrier()
  _exchange(1)

  def _const(i, _):
    deg = sb[i] + pb[i] + 1.0
    dv = _rsqrt16(deg)
    h = tb[i]
    ab[i] = (1.0 - ALPHA) * dv * dv
    bb[i] = ALPHA * dv * h
    tb[i] = dv * h
    return 0
  lax.fori_loop(0, NPT, _const, 0)
  pltpu.sync_copy(tb, T.at[nsl])
  plsc.subcore_barrier()

  def _gstart(ch, buf):
    pltpu.make_async_copy(T.at[rows_v.at[ch]], buf, gsem).start()

  def _gwait(ch, buf):
    pltpu.make_async_copy(T.at[rows_v.at[ch]], buf, gsem).wait()

  def _sstart(ch, buf):
    pltpu.make_async_copy(buf, S.at[cols_v.at[ch]], ssem).start(add=True)

  def _swait(ch, buf):
    pltpu.make_async_copy(buf, S.at[cols_v.at[ch]], ssem).wait()

  def _step(k, carry):
    _gstart(0, msga)

    def _pipe(j, c2):
      chA = 2 * j
      chB = chA + 1
      _gwait(chA, msga)
      _gstart(chB, msgb)
      _sstart(chA, msga)
      _gwait(chB, msgb)
      _swait(chA, msga)

      @pl.when(j < CPT // 2 - 1)
      def _():
        _gstart(chA + 2, msga)

      _sstart(chB, msgb)
      _swait(chB, msgb)
      return c2
    lax.fori_loop(0, CPT // 2, _pipe, 0)
    plsc.subcore_barrier()

    _exchange(k & 1)

    def _upd(i, c2):
      for u in range(4):
        q = i * 4 + u
        tb[q] = ab[q] * (sb[q] + pb[q] + tb[q]) + bb[q]
      return c2
    lax.fori_loop(0, NPT // 4, _upd, 0)
    pltpu.sync_copy(tb, T.at[nsl])
    plsc.subcore_barrier()
    return carry
  lax.fori_loop(0, K_PROP, _step, 0)

  def _fin(i, _):
    sb[i] = tb[i] * _rsqrt16(ab[i] * (1.0 / (1.0 - ALPHA)))
    return 0
  lax.fori_loop(0, NPT, _fin, 0)

  @pl.when(c == 0)
  def _():
    pltpu.sync_copy(sb, out_hbm.at[nsl])


_sc_prop = functools.partial(
    pl.kernel,
    out_type=(
        jax.ShapeDtypeStruct((N_PAD, F), jnp.float32),
        jax.ShapeDtypeStruct((2, N_CORE, N_PAD, F), jnp.float32),
    ),
    mesh=plsc.VectorSubcoreMesh(
        core_axis_name="c", subcore_axis_name="s", num_cores=2),
    compiler_params=pltpu.CompilerParams(
        use_tc_tiling_on_sc=False, needs_layout_passes=False),
    scratch_types=[
        pltpu.VMEM_SHARED((N_PAD, F), jnp.float32),
        pltpu.VMEM_SHARED((N_PAD, F), jnp.float32),
        pltpu.VMEM((NPT, F), jnp.float32),
        pltpu.VMEM((NPT, F), jnp.float32),
        pltpu.VMEM((NPT, F), jnp.float32),
        pltpu.VMEM((NPT, F), jnp.float32),
        pltpu.VMEM((NPT, F), jnp.float32),
        pltpu.VMEM((NPT, F), jnp.float32),
        pltpu.VMEM((CPT, CHUNK), jnp.int32),
        pltpu.VMEM((CPT, CHUNK), jnp.int32),
        pltpu.VMEM((CHUNK, F), jnp.float32),
        pltpu.VMEM((CHUNK, F), jnp.float32),
        pltpu.SemaphoreType.DMA,
        pltpu.SemaphoreType.DMA,
        pltpu.SemaphoreType.REGULAR,
        pltpu.SemaphoreType.DMA,
    ],
)(_sc_body)


def kernel(x, edge_index, W1, b1, W2, b2):
  h = _mlp(x, W1, b1, W2, b2)
  h_pad = jnp.pad(h, ((0, N_PAD - N_NODES), (0, 0)))

  ei = edge_index.astype(jnp.int32)
  pad = jnp.full((E_PAD - E,), N_NODES, jnp.int32)
  rows4 = jnp.concatenate([ei[0], pad]).reshape(N_CORE, N_SUB, CPT, CHUNK)
  cols4 = jnp.concatenate([ei[1], pad]).reshape(N_CORE, N_SUB, CPT, CHUNK)

  out, _ = _sc_prop(rows4, cols4, h_pad)
  return _log_softmax(out[:N_NODES])
